# Initial kernel scaffold; baseline (speedup 1.0000x reference)
import jax
import jax.numpy as jnp
from jax.experimental import pallas as pl


def kernel(x, pe):
    raise NotImplementedError("write your pallas kernel here")


# speedup vs baseline: 3.0199x; 3.0199x over previous
import jax
import jax.numpy as jnp
from jax.experimental import pallas as pl

MAX_POS = 8192
EMBED_DIM = 1024
SEQ_LEN = 8192

_BLOCK_ROWS = 1024


def _copy_block(pe_ref, out_ref):
    out_ref[...] = pe_ref[...]


def kernel(x, pe):
    out = pl.pallas_call(
        _copy_block,
        grid=(MAX_POS // _BLOCK_ROWS,),
        in_specs=[pl.BlockSpec((_BLOCK_ROWS, EMBED_DIM), lambda i: (i, 0))],
        out_specs=pl.BlockSpec((_BLOCK_ROWS, EMBED_DIM), lambda i: (i, 0)),
        out_shape=jax.ShapeDtypeStruct((SEQ_LEN, EMBED_DIM), pe.dtype),
    )(pe)
    return out[None]
